# baseline (speedup 1.0000x reference)
import functools

import jax
import jax.numpy as jnp
from jax import lax
from jax.experimental import pallas as pl
from jax.experimental.pallas import tpu as pltpu
from jax.experimental.pallas import tpu_sc as plsc

BATCH = 16384
N_FIELDS = 26
EMB = 64
B_TOT = BATCH * N_FIELDS
PAIR_D = 2 * EMB
B_PAIR = B_TOT // 2
NC, NS = 2, 16
NW = NC * NS
BPW = B_PAIR // NW
CHUNK = 512
NCHUNK = BPW // CHUNK

_mesh = plsc.VectorSubcoreMesh(core_axis_name="c", subcore_axis_name="s")


@functools.partial(
    pl.kernel,
    mesh=_mesh,
    out_type=jax.ShapeDtypeStruct((B_PAIR, PAIR_D), jnp.float32),
    scratch_types=[
        pltpu.VMEM((CHUNK,), jnp.int32),
        pltpu.VMEM((CHUNK, PAIR_D), jnp.float32),
        pltpu.SemaphoreType.DMA,
    ],
)
def _sc_embed(table_hbm, idx_hbm, out_hbm, idx_v, rows_v, sem):
    wid = lax.axis_index("s") * NC + lax.axis_index("c")
    base0 = wid * BPW

    def body(k, carry):
        base = base0 + k * CHUNK
        pltpu.sync_copy(idx_hbm.at[pl.ds(base, CHUNK)], idx_v)
        pltpu.async_copy(table_hbm.at[idx_v], rows_v, sem).wait()
        pltpu.sync_copy(rows_v, out_hbm.at[pl.ds(base, CHUNK)])
        return carry

    lax.fori_loop(0, NCHUNK, body, 0)


def kernel(ind, ind_emb_weight):
    w = ind_emb_weight
    ptab = jnp.concatenate(
        [jnp.repeat(w, 2, axis=0), jnp.tile(w, (2, 1))], axis=1)
    idx = ind.reshape(B_PAIR, 2).astype(jnp.int32)
    pidx = idx[:, 0] * 2 + idx[:, 1]
    out = _sc_embed(ptab, pidx)
    return out.reshape(BATCH, N_FIELDS, EMB)


# speedup vs baseline: 1.0038x; 1.0038x over previous
import functools

import jax
import jax.numpy as jnp
from jax import lax
from jax.experimental import pallas as pl
from jax.experimental.pallas import tpu as pltpu
from jax.experimental.pallas import tpu_sc as plsc

BATCH = 16384
N_FIELDS = 26
EMB = 64
B_TOT = BATCH * N_FIELDS
PAIR_D = 2 * EMB
B_PAIR = B_TOT // 2
NC, NS = 2, 16
NW = NC * NS
BPW = B_PAIR // NW
CHUNK = 416
NCHUNK = BPW // CHUNK

_mesh = plsc.VectorSubcoreMesh(core_axis_name="c", subcore_axis_name="s")


@functools.partial(
    pl.kernel,
    mesh=_mesh,
    out_type=jax.ShapeDtypeStruct((B_PAIR, PAIR_D), jnp.float32),
    scratch_types=[
        pltpu.VMEM((BPW,), jnp.int32),
        pltpu.VMEM((CHUNK, PAIR_D), jnp.float32),
        pltpu.VMEM((CHUNK, PAIR_D), jnp.float32),
        pltpu.SemaphoreType.DMA,
        pltpu.SemaphoreType.DMA,
        pltpu.SemaphoreType.DMA,
        pltpu.SemaphoreType.DMA,
    ],
)
def _sc_embed(table_hbm, idx_hbm, out_hbm, idx_v, rows0, rows1,
              sg0, sg1, sw0, sw1):
    wid = lax.axis_index("s") * NC + lax.axis_index("c")
    base0 = wid * BPW
    rows = (rows0, rows1)
    sg = (sg0, sg1)
    sw = (sw0, sw1)

    pltpu.sync_copy(idx_hbm.at[pl.ds(base0, BPW)], idx_v)

    def start_gather(k):
        return pltpu.async_copy(
            table_hbm.at[idx_v.at[pl.ds(k * CHUNK, CHUNK)]],
            rows[k % 2], sg[k % 2])

    def start_write(k):
        return pltpu.async_copy(
            rows[k % 2], out_hbm.at[pl.ds(base0 + k * CHUNK, CHUNK)],
            sw[k % 2])

    g = {0: start_gather(0)}
    w = {}
    for k in range(NCHUNK):
        if k + 1 < NCHUNK:
            if k >= 1:
                w[k - 1].wait()
            g[k + 1] = start_gather(k + 1)
        g[k].wait()
        w[k] = start_write(k)
    w[NCHUNK - 2].wait()
    w[NCHUNK - 1].wait()


def kernel(ind, ind_emb_weight):
    w = ind_emb_weight
    ptab = jnp.concatenate(
        [jnp.repeat(w, 2, axis=0), jnp.tile(w, (2, 1))], axis=1)
    idx = ind.reshape(B_PAIR, 2).astype(jnp.int32)
    pidx = idx[:, 0] * 2 + idx[:, 1]
    out = _sc_embed(ptab, pidx)
    return out.reshape(BATCH, N_FIELDS, EMB)


# speedup vs baseline: 3.6470x; 3.6333x over previous
import functools

import jax
import jax.numpy as jnp
from jax import lax
from jax.experimental import pallas as pl
from jax.experimental.pallas import tpu as pltpu
from jax.experimental.pallas import tpu_sc as plsc

BATCH = 16384
N_FIELDS = 26
EMB = 64
B_TOT = BATCH * N_FIELDS
PAIR_D = 2 * EMB
B_PAIR = B_TOT // 2
NC, NS = 2, 16
NW = NC * NS
BPW = B_PAIR // NW
CHUNK = 416
NCHUNK = BPW // CHUNK

_mesh = plsc.VectorSubcoreMesh(core_axis_name="c", subcore_axis_name="s")


@functools.partial(
    pl.kernel,
    mesh=_mesh,
    out_type=jax.ShapeDtypeStruct((B_PAIR, PAIR_D), jnp.float32),
    scratch_types=[
        pltpu.VMEM((BPW,), jnp.int32),
        pltpu.VMEM((CHUNK, PAIR_D), jnp.float32),
        pltpu.VMEM((CHUNK, PAIR_D), jnp.float32),
        pltpu.SemaphoreType.DMA,
        pltpu.SemaphoreType.DMA,
        pltpu.SemaphoreType.DMA,
        pltpu.SemaphoreType.DMA,
    ],
)
def _sc_embed(table_hbm, idx_hbm, out_hbm, idx_v, rows0, rows1,
              sg0, sg1, sw0, sw1):
    wid = lax.axis_index("s") * NC + lax.axis_index("c")
    base0 = wid * BPW
    rows = (rows0, rows1)
    sg = (sg0, sg1)
    sw = (sw0, sw1)

    pltpu.sync_copy(idx_hbm.at[pl.ds(base0, BPW)], idx_v)

    def start_gather(k):
        return pltpu.async_copy(
            table_hbm.at[idx_v.at[pl.ds(k * CHUNK, CHUNK)]],
            rows[k % 2], sg[k % 2])

    def start_write(k):
        return pltpu.async_copy(
            rows[k % 2], out_hbm.at[pl.ds(base0 + k * CHUNK, CHUNK)],
            sw[k % 2])

    g = {0: start_gather(0)}
    w = {}
    for k in range(NCHUNK):
        if k + 1 < NCHUNK:
            if k >= 1:
                w[k - 1].wait()
            g[k + 1] = start_gather(k + 1)
        g[k].wait()
        w[k] = start_write(k)
    w[NCHUNK - 2].wait()
    w[NCHUNK - 1].wait()


def kernel(ind, ind_emb_weight):
    w = ind_emb_weight
    ptab = jnp.concatenate(
        [jnp.repeat(w, 2, axis=0), jnp.tile(w, (2, 1))], axis=1)
    ptab = jnp.tile(ptab, (NW, 1))
    idx = ind.reshape(B_PAIR, 2).astype(jnp.int32)
    pidx = idx[:, 0] * 2 + idx[:, 1]
    pidx = pidx + 4 * (jnp.arange(B_PAIR, dtype=jnp.int32) // BPW)
    out = _sc_embed(ptab, pidx)
    return out.reshape(BATCH, N_FIELDS, EMB)


# speedup vs baseline: 4.8520x; 1.3304x over previous
import functools

import jax
import jax.numpy as jnp
from jax import lax
from jax.experimental import pallas as pl
from jax.experimental.pallas import tpu as pltpu
from jax.experimental.pallas import tpu_sc as plsc

BATCH = 16384
N_FIELDS = 26
EMB = 64
B_TOT = BATCH * N_FIELDS
G = 4
GD = G * EMB
B_G = B_TOT // G
NC, NS = 2, 16
NW = NC * NS
BPW = B_G // NW
CHUNK = 208
NCHUNK = BPW // CHUNK

_mesh = plsc.VectorSubcoreMesh(core_axis_name="c", subcore_axis_name="s")


@functools.partial(
    pl.kernel,
    mesh=_mesh,
    out_type=jax.ShapeDtypeStruct((B_G, GD), jnp.float32),
    scratch_types=[
        pltpu.VMEM((BPW,), jnp.int32),
        pltpu.VMEM((CHUNK, GD), jnp.float32),
        pltpu.VMEM((CHUNK, GD), jnp.float32),
        pltpu.SemaphoreType.DMA,
        pltpu.SemaphoreType.DMA,
        pltpu.SemaphoreType.DMA,
        pltpu.SemaphoreType.DMA,
    ],
)
def _sc_embed(table_hbm, idx_hbm, out_hbm, idx_v, rows0, rows1,
              sg0, sg1, sw0, sw1):
    wid = lax.axis_index("s") * NC + lax.axis_index("c")
    base0 = wid * BPW
    rows = (rows0, rows1)
    sg = (sg0, sg1)
    sw = (sw0, sw1)

    pltpu.sync_copy(idx_hbm.at[pl.ds(base0, BPW)], idx_v)

    def start_gather(k):
        return pltpu.async_copy(
            table_hbm.at[idx_v.at[pl.ds(k * CHUNK, CHUNK)]],
            rows[k % 2], sg[k % 2])

    def start_write(k):
        return pltpu.async_copy(
            rows[k % 2], out_hbm.at[pl.ds(base0 + k * CHUNK, CHUNK)],
            sw[k % 2])

    g = {0: start_gather(0)}
    w = {}
    for k in range(NCHUNK):
        if k + 1 < NCHUNK:
            if k >= 1:
                w[k - 1].wait()
            g[k + 1] = start_gather(k + 1)
        g[k].wait()
        w[k] = start_write(k)
    w[NCHUNK - 2].wait()
    w[NCHUNK - 1].wait()


def kernel(ind, ind_emb_weight):
    w = ind_emb_weight
    e = jnp.arange(2 ** G)
    gtab = jnp.concatenate(
        [w[(e >> (G - 1 - j)) & 1] for j in range(G)], axis=1)
    gtab = jnp.tile(gtab, (NW, 1))
    idx = ind.reshape(B_G, G).astype(jnp.int32)
    gidx = jnp.zeros((B_G,), jnp.int32)
    for j in range(G):
        gidx = gidx * 2 + idx[:, j]
    gidx = gidx + (2 ** G) * (jnp.arange(B_G, dtype=jnp.int32) // BPW)
    out = _sc_embed(gtab, gidx)
    return out.reshape(BATCH, N_FIELDS, EMB)
